# Initial kernel scaffold; baseline (speedup 1.0000x reference)
import jax
import jax.numpy as jnp
from jax.experimental import pallas as pl


def kernel(input, weight, weight_ema):
    raise NotImplementedError("write your pallas kernel here")


# speedup vs baseline: 3.9214x; 3.9214x over previous
import functools

import jax
import jax.numpy as jnp
import numpy as np
from jax import lax
from jax.experimental import pallas as pl
from jax.experimental.pallas import tpu as pltpu
from jax.experimental.pallas import tpu_sc as plsc

DIM = 32
SMOOTHING = np.float32(0.01)
IDXW = 128
CHUNK_IDXROWS = 8


def _gdiff_body(nc, per_w, n_chunks,
                idx_hbm, w_hbm, e_hbm, out_hbm,
                idx_v, w_v, e_v, sem_w, sem_e):
    ch = CHUNK_IDXROWS * IDXW
    wid = lax.axis_index("s") * nc + lax.axis_index("c")
    base = wid * per_w

    half = jnp.float32(0.5)
    three_half = jnp.float32(1.5)
    one = jnp.float32(1.0)

    def chunk_body(c, carry):
        off = pl.multiple_of(base + c * ch, ch)
        idx_row = pl.multiple_of(off // IDXW, CHUNK_IDXROWS)
        pltpu.sync_copy(idx_hbm.at[pl.ds(idx_row, CHUNK_IDXROWS)], idx_v)
        copies = []
        for g in range(CHUNK_IDXROWS):
            dst = pl.ds(g * IDXW, IDXW)
            copies.append(pltpu.async_copy(w_hbm.at[idx_v.at[g]],
                                           w_v.at[dst], sem_w))
            copies.append(pltpu.async_copy(e_hbm.at[idx_v.at[g]],
                                           e_v.at[dst], sem_e))
        for cp in copies:
            cp.wait()

        def row_body(i, carry2):
            for j in (0, 16):
                w = w_v[i, pl.ds(j, 16)]
                e = e_v[i, pl.ds(j, 16)]
                e2 = e + SMOOTHING * (w - e)
                p = w * e2
                a = jnp.abs(p)
                bi = lax.bitcast_convert_type(a, jnp.int32)
                bi = 0x5F3759DF - lax.shift_right_logical(bi, 1)
                r = lax.bitcast_convert_type(bi, jnp.float32)
                r = r * (three_half - half * a * r * r)
                r = r * (three_half - half * a * r * r)
                mag = a * r
                w_v[i, pl.ds(j, 16)] = jnp.sign(w + e2) * mag
            return carry2

        lax.fori_loop(0, ch, row_body, 0, unroll=4)
        pltpu.sync_copy(w_v, out_hbm.at[pl.ds(off, ch)])
        return carry

    lax.fori_loop(0, n_chunks, chunk_body, 0)


def kernel(input, weight, weight_ema):
    b = input.size
    idx2d = input.reshape(b // IDXW, IDXW).astype(jnp.int32)
    info = plsc.get_sparse_core_info()
    nc, ns = info.num_cores, info.num_subcores
    nw = nc * ns
    ch = CHUNK_IDXROWS * IDXW
    assert b % (nw * ch) == 0, (b, nw, ch)
    per_w = b // nw
    n_chunks = per_w // ch

    mesh = plsc.VectorSubcoreMesh(core_axis_name="c", subcore_axis_name="s")
    run = pl.kernel(
        functools.partial(_gdiff_body, nc, per_w, n_chunks),
        mesh=mesh,
        compiler_params=pltpu.CompilerParams(use_tc_tiling_on_sc=False),
        out_type=jax.ShapeDtypeStruct((b, DIM), jnp.float32),
        scratch_types=[
            pltpu.VMEM((CHUNK_IDXROWS, IDXW), jnp.int32),
            pltpu.VMEM((ch, DIM), jnp.float32),
            pltpu.VMEM((ch, DIM), jnp.float32),
            pltpu.SemaphoreType.DMA,
            pltpu.SemaphoreType.DMA,
        ],
    )
    out = run(idx2d, weight, weight_ema)
    return out.reshape(input.shape + (DIM,))


# speedup vs baseline: 5.6267x; 1.4349x over previous
import functools

import jax
import jax.numpy as jnp
import numpy as np
from jax import lax
from jax.experimental import pallas as pl
from jax.experimental.pallas import tpu as pltpu
from jax.experimental.pallas import tpu_sc as plsc

DIM = 32
SMOOTHING = np.float32(0.01)
CT = 8


def _gdiff_body(nc, t_per_w, n_chunks, ncols,
                idx_hbm, w_hbm, e_hbm, out_hbm,
                idx_all, w0, w1, e0, e1, o0, o1,
                sem_w0, sem_w1, sem_e0, sem_e1):
    wid = lax.axis_index("s") * nc + lax.axis_index("c")
    t_base = pl.multiple_of(wid * t_per_w, t_per_w)

    w_bufs = (w0, w1)
    e_bufs = (e0, e1)
    o_bufs = (o0, o1)
    sems_w = (sem_w0, sem_w1)
    sems_e = (sem_e0, sem_e1)

    pltpu.sync_copy(idx_hbm.at[pl.ds(t_base, t_per_w)], idx_all)

    def fire(c, k):
        for t in range(CT):
            row = c * CT + t
            pltpu.async_copy(w_hbm.at[idx_all.at[row]], w_bufs[k].at[t],
                             sems_w[k])
            pltpu.async_copy(e_hbm.at[idx_all.at[row]], e_bufs[k].at[t],
                             sems_e[k])

    def drain(k):
        pltpu.make_async_copy(out_hbm.at[pl.ds(0, CT)], w_bufs[k],
                              sems_w[k]).wait()
        pltpu.make_async_copy(out_hbm.at[pl.ds(0, CT)], e_bufs[k],
                              sems_e[k]).wait()

    half = jnp.float32(0.5)
    three_half = jnp.float32(1.5)
    sign_mask = jnp.int32(-2147483648)

    def compute(k):
        wv, ev, ov = w_bufs[k], e_bufs[k], o_bufs[k]

        def col_body(j, carry):
            for t in range(CT):
                for h in (0, 16):
                    w = wv[t, j, pl.ds(h, 16)]
                    e = ev[t, j, pl.ds(h, 16)]
                    e2 = e + SMOOTHING * (w - e)
                    p = w * e2
                    a = jnp.abs(p)
                    bi = lax.bitcast_convert_type(a, jnp.int32)
                    bi = 0x5F3759DF - lax.shift_right_logical(bi, 1)
                    r = lax.bitcast_convert_type(bi, jnp.float32)
                    r = r * (three_half - half * a * r * r)
                    mag = a * r
                    sb = lax.bitcast_convert_type(w + e2, jnp.int32) & sign_mask
                    ob = lax.bitcast_convert_type(mag, jnp.int32) ^ sb
                    ov[t, j, pl.ds(h, 16)] = lax.bitcast_convert_type(
                        ob, jnp.float32)
            return carry

        lax.fori_loop(0, ncols, col_body, 0, unroll=2)

    fire(0, 0)

    def pair_body(c0, carry):
        for k in (0, 1):
            c = c0 * 2 + k

            @pl.when(c + 1 < n_chunks)
            def _():
                fire(c + 1, 1 - k)

            drain(k)
            compute(k)
            tb = pl.multiple_of(t_base + c * CT, CT)
            pltpu.sync_copy(o_bufs[k], out_hbm.at[pl.ds(tb, CT)])
        return carry

    lax.fori_loop(0, n_chunks // 2, pair_body, 0)


def kernel(input, weight, weight_ema):
    nrows, ncols = input.shape
    idx2d = input.astype(jnp.int32)
    info = plsc.get_sparse_core_info()
    nc, ns = info.num_cores, info.num_subcores
    nw = nc * ns
    t_per_w = nrows // nw
    n_chunks = t_per_w // CT
    assert nrows % (nw * 2 * CT) == 0, (nrows, nw, CT)

    mesh = plsc.VectorSubcoreMesh(core_axis_name="c", subcore_axis_name="s")
    run = pl.kernel(
        functools.partial(_gdiff_body, nc, t_per_w, n_chunks, ncols),
        mesh=mesh,
        compiler_params=pltpu.CompilerParams(use_tc_tiling_on_sc=False),
        out_type=jax.ShapeDtypeStruct((nrows, ncols, DIM), jnp.float32),
        scratch_types=[
            pltpu.VMEM((t_per_w, ncols), jnp.int32),
            pltpu.VMEM((CT, ncols, DIM), jnp.float32),
            pltpu.VMEM((CT, ncols, DIM), jnp.float32),
            pltpu.VMEM((CT, ncols, DIM), jnp.float32),
            pltpu.VMEM((CT, ncols, DIM), jnp.float32),
            pltpu.VMEM((CT, ncols, DIM), jnp.float32),
            pltpu.VMEM((CT, ncols, DIM), jnp.float32),
            pltpu.SemaphoreType.DMA,
            pltpu.SemaphoreType.DMA,
            pltpu.SemaphoreType.DMA,
            pltpu.SemaphoreType.DMA,
        ],
    )
    return run(idx2d, weight, weight_ema)


# speedup vs baseline: 7.8912x; 1.4024x over previous
import functools

import jax
import jax.numpy as jnp
import numpy as np
from jax import lax
from jax.experimental import pallas as pl
from jax.experimental.pallas import tpu as pltpu
from jax.experimental.pallas import tpu_sc as plsc

DIM = 32
SMOOTHING = np.float32(0.01)
CT = 8


def _gdiff_body(nc, t_per_w, n_chunks, ncols,
                idx_hbm, w_hbm, e_hbm, out_hbm,
                idx_all, w0, w1, e0, e1, o0, o1,
                sem_w0, sem_w1, sem_e0, sem_e1):
    wid = lax.axis_index("s") * nc + lax.axis_index("c")
    t_base = pl.multiple_of(wid * t_per_w, t_per_w)

    w_bufs = (w0, w1)
    e_bufs = (e0, e1)
    o_bufs = (o0, o1)
    sems_w = (sem_w0, sem_w1)
    sems_e = (sem_e0, sem_e1)

    pltpu.sync_copy(idx_hbm.at[pl.ds(t_base, t_per_w)], idx_all)

    def fire(c, k):
        for t in range(CT):
            row = c * CT + t
            pltpu.async_copy(w_hbm.at[idx_all.at[row]], w_bufs[k].at[t],
                             sems_w[k])
            pltpu.async_copy(e_hbm.at[idx_all.at[row]], e_bufs[k].at[t],
                             sems_e[k])

    def drain(k):
        pltpu.make_async_copy(out_hbm.at[pl.ds(0, CT)], w_bufs[k],
                              sems_w[k]).wait()
        pltpu.make_async_copy(out_hbm.at[pl.ds(0, CT)], e_bufs[k],
                              sems_e[k]).wait()

    half = jnp.float32(0.5)
    three_half = jnp.float32(1.5)
    sign_mask = jnp.int32(-2147483648)

    def compute(k):
        wv, ev, ov = w_bufs[k], e_bufs[k], o_bufs[k]

        @plsc.parallel_loop(0, ncols, unroll=2)
        def col_body(j):
            for t in range(CT):
                for h in (0, 16):
                    w = wv[t, j, pl.ds(h, 16)]
                    e = ev[t, j, pl.ds(h, 16)]
                    e2 = e + SMOOTHING * (w - e)
                    p = w * e2
                    a = jnp.abs(p)
                    bi = lax.bitcast_convert_type(a, jnp.int32)
                    bi = 0x5F3759DF - lax.shift_right_logical(bi, 1)
                    r = lax.bitcast_convert_type(bi, jnp.float32)
                    r = r * (three_half - half * a * r * r)
                    mag = a * r
                    sb = lax.bitcast_convert_type(w + e2, jnp.int32) & sign_mask
                    ob = lax.bitcast_convert_type(mag, jnp.int32) ^ sb
                    ov[t, j, pl.ds(h, 16)] = lax.bitcast_convert_type(
                        ob, jnp.float32)

    fire(0, 0)

    def pair_body(c0, carry):
        for k in (0, 1):
            c = c0 * 2 + k

            @pl.when(c + 1 < n_chunks)
            def _():
                fire(c + 1, 1 - k)

            drain(k)
            compute(k)
            tb = pl.multiple_of(t_base + c * CT, CT)
            pltpu.sync_copy(o_bufs[k], out_hbm.at[pl.ds(tb, CT)])
        return carry

    lax.fori_loop(0, n_chunks // 2, pair_body, 0)


def kernel(input, weight, weight_ema):
    nrows, ncols = input.shape
    idx2d = input.astype(jnp.int32)
    info = plsc.get_sparse_core_info()
    nc, ns = info.num_cores, info.num_subcores
    nw = nc * ns
    t_per_w = nrows // nw
    n_chunks = t_per_w // CT
    assert nrows % (nw * 2 * CT) == 0, (nrows, nw, CT)

    mesh = plsc.VectorSubcoreMesh(core_axis_name="c", subcore_axis_name="s")
    run = pl.kernel(
        functools.partial(_gdiff_body, nc, t_per_w, n_chunks, ncols),
        mesh=mesh,
        compiler_params=pltpu.CompilerParams(use_tc_tiling_on_sc=False),
        out_type=jax.ShapeDtypeStruct((nrows, ncols, DIM), jnp.float32),
        scratch_types=[
            pltpu.VMEM((t_per_w, ncols), jnp.int32),
            pltpu.VMEM((CT, ncols, DIM), jnp.float32),
            pltpu.VMEM((CT, ncols, DIM), jnp.float32),
            pltpu.VMEM((CT, ncols, DIM), jnp.float32),
            pltpu.VMEM((CT, ncols, DIM), jnp.float32),
            pltpu.VMEM((CT, ncols, DIM), jnp.float32),
            pltpu.VMEM((CT, ncols, DIM), jnp.float32),
            pltpu.SemaphoreType.DMA,
            pltpu.SemaphoreType.DMA,
            pltpu.SemaphoreType.DMA,
            pltpu.SemaphoreType.DMA,
        ],
    )
    return run(idx2d, weight, weight_ema)


# speedup vs baseline: 7.9007x; 1.0012x over previous
import functools

import jax
import jax.numpy as jnp
import numpy as np
from jax import lax
from jax.experimental import pallas as pl
from jax.experimental.pallas import tpu as pltpu
from jax.experimental.pallas import tpu_sc as plsc

DIM = 32
SMOOTHING = np.float32(0.01)
CT = 8


def _gdiff_body(nc, t_per_w, n_chunks, ncols,
                idx_hbm, w_hbm, e_hbm, out_hbm,
                idx_all, w0, w1, e0, e1, o0, o1,
                sem_w0, sem_w1, sem_e0, sem_e1):
    wid = lax.axis_index("s") * nc + lax.axis_index("c")
    t_base = pl.multiple_of(wid * t_per_w, t_per_w)
    row_ct = CT * ncols

    w_bufs = (w0, w1)
    e_bufs = (e0, e1)
    o_bufs = (o0, o1)
    sems_w = (sem_w0, sem_w1)
    sems_e = (sem_e0, sem_e1)

    pltpu.sync_copy(idx_hbm.at[pl.ds(t_base, t_per_w)], idx_all)

    def fire(c, k):
        for t in range(CT):
            row = c * CT + t
            dst = pl.ds(t * ncols, ncols)
            pltpu.async_copy(w_hbm.at[idx_all.at[row]], w_bufs[k].at[dst],
                             sems_w[k])
            pltpu.async_copy(e_hbm.at[idx_all.at[row]], e_bufs[k].at[dst],
                             sems_e[k])

    def drain(k):
        pltpu.make_async_copy(w_hbm.at[pl.ds(0, row_ct)], w_bufs[k],
                              sems_w[k]).wait()
        pltpu.make_async_copy(e_hbm.at[pl.ds(0, row_ct)], e_bufs[k],
                              sems_e[k]).wait()

    half = jnp.float32(0.5)
    three_half = jnp.float32(1.5)
    sign_mask = jnp.int32(-2147483648)

    def compute(k):
        wv, ev, ov = w_bufs[k], e_bufs[k], o_bufs[k]

        @plsc.parallel_loop(0, ncols, unroll=2)
        def col_body(j):
            for t in range(CT):
                row = t * ncols + j
                for h in (0, 16):
                    w = wv[row, pl.ds(h, 16)]
                    e = ev[row, pl.ds(h, 16)]
                    e2 = e + SMOOTHING * (w - e)
                    p = w * e2
                    a = jnp.abs(p)
                    bi = lax.bitcast_convert_type(a, jnp.int32)
                    bi = 0x5F3759DF - lax.shift_right_logical(bi, 1)
                    r = lax.bitcast_convert_type(bi, jnp.float32)
                    r = r * (three_half - half * a * r * r)
                    mag = a * r
                    sb = lax.bitcast_convert_type(w + e2, jnp.int32) & sign_mask
                    ob = lax.bitcast_convert_type(mag, jnp.int32) ^ sb
                    ov[pl.ds(row * DIM + h, 16)] = lax.bitcast_convert_type(
                        ob, jnp.float32)

    fire(0, 0)

    def pair_body(c0, carry):
        for k in (0, 1):
            c = c0 * 2 + k

            @pl.when(c + 1 < n_chunks)
            def _():
                fire(c + 1, 1 - k)

            drain(k)
            compute(k)
            ob = pl.multiple_of((t_base + c * CT) * ncols * DIM,
                                row_ct * DIM)
            pltpu.sync_copy(o_bufs[k], out_hbm.at[pl.ds(ob, row_ct * DIM)])
        return carry

    lax.fori_loop(0, n_chunks // 2, pair_body, 0)


def kernel(input, weight, weight_ema):
    nrows, ncols = input.shape
    idx2d = input.astype(jnp.int32)
    info = plsc.get_sparse_core_info()
    nc, ns = info.num_cores, info.num_subcores
    nw = nc * ns
    t_per_w = nrows // nw
    n_chunks = t_per_w // CT
    assert nrows % (nw * 2 * CT) == 0, (nrows, nw, CT)

    mesh = plsc.VectorSubcoreMesh(core_axis_name="c", subcore_axis_name="s")
    run = pl.kernel(
        functools.partial(_gdiff_body, nc, t_per_w, n_chunks, ncols),
        mesh=mesh,
        compiler_params=pltpu.CompilerParams(use_tc_tiling_on_sc=False),
        out_type=jax.ShapeDtypeStruct((nrows * ncols * DIM,), jnp.float32),
        scratch_types=[
            pltpu.VMEM((t_per_w, ncols), jnp.int32),
            pltpu.VMEM((CT * ncols, DIM), jnp.float32),
            pltpu.VMEM((CT * ncols, DIM), jnp.float32),
            pltpu.VMEM((CT * ncols, DIM), jnp.float32),
            pltpu.VMEM((CT * ncols, DIM), jnp.float32),
            pltpu.VMEM((CT * ncols * DIM,), jnp.float32),
            pltpu.VMEM((CT * ncols * DIM,), jnp.float32),
            pltpu.SemaphoreType.DMA,
            pltpu.SemaphoreType.DMA,
            pltpu.SemaphoreType.DMA,
            pltpu.SemaphoreType.DMA,
        ],
    )
    flat = run(idx2d, weight, weight_ema)
    return flat.reshape(nrows, ncols, DIM)
